# baseline (speedup 1.0000x reference)
import functools

import jax
import jax.numpy as jnp
from jax import lax
from jax.experimental import pallas as pl
from jax.experimental.pallas import tpu as pltpu
from jax.experimental.pallas import tpu_sc as plsc

B, T, V, E, H = 4096, 20, 100000, 64, 64
BT = B * T
TP = T // 2

NC, NS = 2, 16
NW = NC * NS
PER_W = BT // NW
GROW = 128
NG = PER_W // GROW
MACRO = 1280
GPM = MACRO // GROW
NMACRO = PER_W // MACRO


def _sc_gather(q3, oidx3, emb_table):
    mesh = plsc.VectorSubcoreMesh(core_axis_name="c", subcore_axis_name="s")

    @functools.partial(
        pl.kernel,
        mesh=mesh,
        out_type=jax.ShapeDtypeStruct((BT, E), jnp.float32),
        compiler_params=pltpu.CompilerParams(use_tc_tiling_on_sc=False),
        scratch_types=[
            pltpu.VMEM((NG, GROW), jnp.int32),
            pltpu.VMEM((NG, GROW), jnp.int32),
            pltpu.VMEM((MACRO, E), jnp.float32),
            pltpu.SemaphoreType.DMA,
            pltpu.SemaphoreType.DMA,
        ],
    )
    def gather_kernel(idx_hbm, oidx_hbm, table_hbm, out_hbm, idx_v, oidx_v,
                      rows_v, gsem, ssem):
        wid = lax.axis_index("s") * NC + lax.axis_index("c")
        out_rows = out_hbm
        pltpu.sync_copy(idx_hbm.at[wid], idx_v)
        pltpu.sync_copy(oidx_hbm.at[wid], oidx_v)
        for c in range(NMACRO):
            gathers = []
            for j in range(GPM):
                g = c * GPM + j
                gathers.append(
                    pltpu.async_copy(
                        table_hbm.at[idx_v.at[g]],
                        rows_v.at[pl.ds(j * GROW, GROW)],
                        gsem,
                    )
                )
            for cp in gathers:
                cp.wait()
            scatters = []
            for j in range(GPM):
                g = c * GPM + j
                scatters.append(
                    pltpu.async_copy(
                        rows_v.at[pl.ds(j * GROW, GROW)],
                        out_rows.at[oidx_v.at[g]],
                        ssem,
                    )
                )
            for cp in scatters:
                cp.wait()

    return gather_kernel(q3, oidx3, emb_table)


BBLK = 4096


def _gru_body(emb_ref, wpair_ref, whh_ref, bpair_ref, bhh_ref, out_ref):
    wpair = wpair_ref[...]
    whh = whh_ref[...]
    bpair = bpair_ref[...]
    bhh = bhh_ref[...]
    h = jnp.zeros((BBLK, H), dtype=jnp.float32)
    for u in range(TP):
        x2 = emb_ref[u]
        gi2 = jnp.dot(x2, wpair, preferred_element_type=jnp.float32) + bpair
        for t_lo in range(2):
            gi = gi2[:, t_lo * 3 * H:(t_lo + 1) * 3 * H]
            gh = jnp.dot(h, whh, preferred_element_type=jnp.float32) + bhh
            s = jax.nn.sigmoid(gi[:, :2 * H] + gh[:, :2 * H])
            r = s[:, :H]
            z = s[:, H:]
            n = jnp.tanh(gi[:, 2 * H:] + r * gh[:, 2 * H:])
            h = (1.0 - z) * n + z * h
    out_ref[...] = h


def _gru(emb3, wpair, whh, bpair, bhh):
    return pl.pallas_call(
        _gru_body,
        grid=(B // BBLK,),
        in_specs=[
            pl.BlockSpec((TP, BBLK, 2 * E), lambda i: (0, i, 0)),
            pl.BlockSpec((2 * E, 6 * H), lambda i: (0, 0)),
            pl.BlockSpec((H, 3 * H), lambda i: (0, 0)),
            pl.BlockSpec((1, 6 * H), lambda i: (0, 0)),
            pl.BlockSpec((1, 3 * H), lambda i: (0, 0)),
        ],
        out_specs=pl.BlockSpec((BBLK, H), lambda i: (i, 0)),
        out_shape=jax.ShapeDtypeStruct((B, H), jnp.float32),
    )(emb3, wpair, whh, bpair, bhh)


def kernel(question, question_lengths, pack_sequence, emb_table, W_ih, W_hh, b_ih, b_hh):
    q3 = question.astype(jnp.int32).reshape(NW, NG, GROW)
    f = jnp.arange(BT, dtype=jnp.int32)
    oidx = ((f % T) // 2) * (2 * B) + (f // T) * 2 + (f % 2)
    oidx3 = oidx.reshape(NW, NG, GROW)
    emb = _sc_gather(q3, oidx3, emb_table)
    emb3 = emb.reshape(TP, B, 2 * E)
    wihT = W_ih.T
    zeros = jnp.zeros_like(wihT)
    wpair = jnp.concatenate(
        [jnp.concatenate([wihT, zeros], axis=1),
         jnp.concatenate([zeros, wihT], axis=1)], axis=0)
    bpair = jnp.concatenate([b_ih, b_ih]).reshape(1, 6 * H)
    return _gru(emb3, wpair, W_hh.T, bpair, b_hh.reshape(1, 3 * H))


# speedup vs baseline: 1.0297x; 1.0297x over previous
import functools

import jax
import jax.numpy as jnp
from jax import lax
from jax.experimental import pallas as pl
from jax.experimental.pallas import tpu as pltpu
from jax.experimental.pallas import tpu_sc as plsc

B, T, V, E, H = 4096, 20, 100000, 64, 64
BT = B * T
TP = T // 2

NC, NS = 2, 16
NW = NC * NS
PER_W = BT // NW
GROW = 128
MACRO = 1280
GPM = MACRO // GROW
NMACRO = PER_W // MACRO
WSEG = 256
SPM = MACRO // WSEG


def _sc_gather(qperm, emb_table):
    mesh = plsc.VectorSubcoreMesh(core_axis_name="c", subcore_axis_name="s")

    @functools.partial(
        pl.kernel,
        mesh=mesh,
        out_type=jax.ShapeDtypeStruct((TP, B, 2 * E), jnp.float32),
        compiler_params=pltpu.CompilerParams(use_tc_tiling_on_sc=False),
        scratch_types=[
            pltpu.VMEM((PER_W,), jnp.int32),
            pltpu.VMEM((MACRO, E), jnp.float32),
            pltpu.SemaphoreType.DMA,
        ],
    )
    def gather_kernel(idx_hbm, table_hbm, out_hbm, idx_v, rows_v, sem):
        wid = lax.axis_index("s") * NC + lax.axis_index("c")
        t_lo = wid // (NW // 2)
        base = (wid % (NW // 2)) * PER_W
        pltpu.sync_copy(idx_hbm.at[t_lo, pl.ds(base, PER_W)], idx_v)
        for c in range(NMACRO):
            copies = []
            for j in range(GPM):
                g = c * GPM + j
                copies.append(
                    pltpu.async_copy(
                        table_hbm.at[idx_v.at[pl.ds(g * GROW, GROW)]],
                        rows_v.at[pl.ds(j * GROW, GROW)],
                        sem,
                    )
                )
            for cp in copies:
                cp.wait()
            for s in range(SPM):
                r = base + c * MACRO + s * WSEG
                u = r // B
                b0 = r % B
                pltpu.sync_copy(
                    rows_v.at[pl.ds(s * WSEG, WSEG)],
                    out_hbm.at[u, pl.ds(b0, WSEG), pl.ds(t_lo * E, E)],
                )

    return gather_kernel(qperm, emb_table)


BBLK = 1024


def _gru_body(emb_ref, wpair_ref, whh_ref, bpair_ref, bhh_ref, out_ref):
    wpair = wpair_ref[...]
    whh = whh_ref[...]
    bpair = bpair_ref[...]
    bhh = bhh_ref[...]
    h = jnp.zeros((BBLK, H), dtype=jnp.float32)
    for u in range(TP):
        x2 = emb_ref[u]
        gi2 = jnp.dot(x2, wpair, preferred_element_type=jnp.float32) + bpair
        for t_lo in range(2):
            gi = gi2[:, t_lo * 3 * H:(t_lo + 1) * 3 * H]
            gh = jnp.dot(h, whh, preferred_element_type=jnp.float32) + bhh
            s = jax.nn.sigmoid(gi[:, :2 * H] + gh[:, :2 * H])
            r = s[:, :H]
            z = s[:, H:]
            n = jnp.tanh(gi[:, 2 * H:] + r * gh[:, 2 * H:])
            h = (1.0 - z) * n + z * h
    out_ref[...] = h


def _gru(emb3, wpair, whh, bpair, bhh):
    return pl.pallas_call(
        _gru_body,
        grid=(B // BBLK,),
        in_specs=[
            pl.BlockSpec((TP, BBLK, 2 * E), lambda i: (0, i, 0)),
            pl.BlockSpec((2 * E, 6 * H), lambda i: (0, 0)),
            pl.BlockSpec((H, 3 * H), lambda i: (0, 0)),
            pl.BlockSpec((1, 6 * H), lambda i: (0, 0)),
            pl.BlockSpec((1, 3 * H), lambda i: (0, 0)),
        ],
        out_specs=pl.BlockSpec((BBLK, H), lambda i: (i, 0)),
        out_shape=jax.ShapeDtypeStruct((B, H), jnp.float32),
    )(emb3, wpair, whh, bpair, bhh)


def kernel(question, question_lengths, pack_sequence, emb_table, W_ih, W_hh, b_ih, b_hh):
    qperm = (
        question.astype(jnp.int32)
        .reshape(B, TP, 2)
        .transpose(2, 1, 0)
        .reshape(2, TP * B)
    )
    emb3 = _sc_gather(qperm, emb_table)
    wihT = W_ih.T
    zeros = jnp.zeros_like(wihT)
    wpair = jnp.concatenate(
        [jnp.concatenate([wihT, zeros], axis=1),
         jnp.concatenate([zeros, wihT], axis=1)], axis=0)
    bpair = jnp.concatenate([b_ih, b_ih]).reshape(1, 6 * H)
    return _gru(emb3, wpair, W_hh.T, bpair, b_hh.reshape(1, 3 * H))


# speedup vs baseline: 1.0324x; 1.0027x over previous
import functools

import jax
import jax.numpy as jnp
from jax import lax
from jax.experimental import pallas as pl
from jax.experimental.pallas import tpu as pltpu
from jax.experimental.pallas import tpu_sc as plsc

B, T, V, E, H = 4096, 20, 100000, 64, 64
BT = B * T
TP = T // 2

NC, NS = 2, 16
NW = NC * NS
PER_W = BT // NW
GROW = 128
MACRO = 1280
GPM = MACRO // GROW
NMACRO = PER_W // MACRO
WSEG = 256
SPM = MACRO // WSEG


def _sc_gather(qperm, emb_table):
    mesh = plsc.VectorSubcoreMesh(core_axis_name="c", subcore_axis_name="s")

    @functools.partial(
        pl.kernel,
        mesh=mesh,
        out_type=jax.ShapeDtypeStruct((TP, B, 2 * E), jnp.float32),
        compiler_params=pltpu.CompilerParams(use_tc_tiling_on_sc=False),
        scratch_types=[
            pltpu.VMEM((PER_W,), jnp.int32),
            pltpu.VMEM((MACRO, E), jnp.float32),
            pltpu.SemaphoreType.DMA,
        ],
    )
    def gather_kernel(idx_hbm, table_hbm, out_hbm, idx_v, rows_v, sem):
        wid = lax.axis_index("s") * NC + lax.axis_index("c")
        t_lo = wid // (NW // 2)
        base = (wid % (NW // 2)) * PER_W
        pltpu.sync_copy(idx_hbm.at[t_lo, pl.ds(base, PER_W)], idx_v)
        for c in range(NMACRO):
            copies = []
            for j in range(GPM):
                g = c * GPM + j
                copies.append(
                    pltpu.async_copy(
                        table_hbm.at[idx_v.at[pl.ds(g * GROW, GROW)]],
                        rows_v.at[pl.ds(j * GROW, GROW)],
                        sem,
                    )
                )
            for cp in copies:
                cp.wait()
            for s in range(SPM):
                r = base + c * MACRO + s * WSEG
                u = r // B
                b0 = r % B
                pltpu.sync_copy(
                    rows_v.at[pl.ds(s * WSEG, WSEG)],
                    out_hbm.at[u, pl.ds(b0, WSEG), pl.ds(t_lo * E, E)],
                )

    return gather_kernel(qperm, emb_table)


BBLK = 1024


def _gru_body(emb_ref, wpair_ref, whh_ref, bpair_ref, bhh_ref, out_ref):
    wpair = wpair_ref[...]
    whh = whh_ref[...]
    bpair = bpair_ref[...]
    bhh = bhh_ref[...]
    h = jnp.zeros((BBLK, H), dtype=jnp.float32)
    for u in range(TP):
        x2 = emb_ref[u]
        gi2 = jnp.dot(x2, wpair, preferred_element_type=jnp.float32) + bpair
        for t_lo in range(2):
            gi = gi2[:, t_lo * 3 * H:(t_lo + 1) * 3 * H]
            gh = jnp.dot(h, whh, preferred_element_type=jnp.float32) + bhh
            s = jax.nn.sigmoid(gi[:, :2 * H] + gh[:, :2 * H])
            r = s[:, :H]
            z = s[:, H:]
            n = jnp.tanh(gi[:, 2 * H:] + r * gh[:, 2 * H:])
            h = (1.0 - z) * n + z * h
    out_ref[...] = h


def _gru(emb3, wpair, whh, bpair, bhh):
    return pl.pallas_call(
        _gru_body,
        grid=(B // BBLK,),
        in_specs=[
            pl.BlockSpec((TP, BBLK, 2 * E), lambda i: (0, i, 0)),
            pl.BlockSpec((2 * E, 6 * H), lambda i: (0, 0)),
            pl.BlockSpec((H, 3 * H), lambda i: (0, 0)),
            pl.BlockSpec((1, 6 * H), lambda i: (0, 0)),
            pl.BlockSpec((1, 3 * H), lambda i: (0, 0)),
        ],
        out_specs=pl.BlockSpec((BBLK, H), lambda i: (i, 0)),
        out_shape=jax.ShapeDtypeStruct((B, H), jnp.float32),
    )(emb3, wpair, whh, bpair, bhh)


def kernel(question, question_lengths, pack_sequence, emb_table, W_ih, W_hh, b_ih, b_hh):
    qperm = (
        question.astype(jnp.int32)
        .T.reshape(TP, 2, B)
        .transpose(1, 0, 2)
        .reshape(2, TP * B)
    )
    emb3 = _sc_gather(qperm, emb_table)
    wihT = W_ih.T
    zeros = jnp.zeros_like(wihT)
    wpair = jnp.concatenate(
        [jnp.concatenate([wihT, zeros], axis=1),
         jnp.concatenate([zeros, wihT], axis=1)], axis=0)
    bpair = jnp.concatenate([b_ih, b_ih]).reshape(1, 6 * H)
    return _gru(emb3, wpair, W_hh.T, bpair, b_hh.reshape(1, 3 * H))
